# baseline (speedup 1.0000x reference)
import functools

import jax
import jax.numpy as jnp
from jax import lax
from jax.experimental import pallas as pl
from jax.experimental.pallas import tpu as pltpu
from jax.experimental.pallas import tpu_sc as plsc

NC, NS, L = 2, 16, 16
NW = NC * NS
BSZ, SEQ, NA, D = 2, 2048, 8, 1024
ROWS = BSZ * SEQ

S_SC = 1024
TC_S = SEQ - S_SC
SC_ROWS = BSZ * S_SC
RPW = SC_ROWS // NW
CH = 16
NBUF = 3
NCHUNK = RPW // CH
WPB = NW // BSZ
SPW = S_SC // WPB


def _body(ao, res, idxb, out, idx_v, idx_vv, *bufs_and_sems):
    gbuf = bufs_and_sems[0:NBUF]
    rbuf = bufs_and_sems[NBUF:2 * NBUF]
    gsem = bufs_and_sems[2 * NBUF:3 * NBUF]
    rsem = bufs_and_sems[3 * NBUF:4 * NBUF]
    osem = bufs_and_sems[4 * NBUF:5 * NBUF]

    c = lax.axis_index("c")
    s = lax.axis_index("s")
    wid = s * NC + c
    b = wid // WPB
    s0 = (wid % WPB) * SPW
    gbase = b * SEQ + s0
    obase = b * S_SC + s0

    pltpu.sync_copy(idxb.at[b], idx_vv)
    iv = idx_vv[...]

    for j in range(RPW // L):
        idx_v[pl.ds(j * L, L)] = (
            gbase + j * L + lax.iota(jnp.int32, L)) * NA + iv

    def issue_in(ch):
        nb = ch % NBUF
        g = pltpu.async_copy(
            ao.at[idx_v.at[pl.ds(ch * CH, CH)]], gbuf[nb], gsem[nb])
        r = pltpu.async_copy(
            res.at[pl.ds(gbase + ch * CH, CH)], rbuf[nb], rsem[nb])
        return g, r

    pend_in = {}
    pend_out = {}
    for ch in range(min(NBUF - 1, NCHUNK)):
        pend_in[ch] = issue_in(ch)
    for ch in range(NCHUNK):
        nb = ch % NBUF
        ahead = ch + NBUF - 1
        if ahead < NCHUNK:
            if ahead - NBUF in pend_out:
                pend_out.pop(ahead - NBUF).wait()
            pend_in[ahead] = issue_in(ahead)
        g, r = pend_in.pop(ch)
        g.wait()
        r.wait()

        def row_add(i, carry, _g=gbuf[nb], _r=rbuf[nb]):
            for j in range(D // L):
                sl = pl.ds(j * L, L)
                plsc.addupdate(_r.at[i, sl], _g[i, sl])
            return carry

        lax.fori_loop(0, CH, row_add, 0)
        pend_out[ch] = pltpu.async_copy(
            rbuf[nb], out.at[pl.ds(obase + ch * CH, CH)], osem[nb])
    for cp in pend_out.values():
        cp.wait()


_sc_call = functools.partial(
    pl.kernel,
    out_type=jax.ShapeDtypeStruct((SC_ROWS, D), jnp.float32),
    mesh=plsc.VectorSubcoreMesh(core_axis_name="c", subcore_axis_name="s"),
    scratch_types=(
        [pltpu.VMEM((RPW,), jnp.int32),
         pltpu.VMEM((L,), jnp.int32)]
        + [pltpu.VMEM((CH, D), jnp.float32)] * (2 * NBUF)
        + [pltpu.SemaphoreType.DMA] * (3 * NBUF)
    ),
)(_body)


BS = 256


def _tc_body(idx_ref, a_ref, r_ref, o_ref):
    o_ref[...] = a_ref[...] + r_ref[...]


_tc_call = pl.pallas_call(
    _tc_body,
    grid_spec=pltpu.PrefetchScalarGridSpec(
        num_scalar_prefetch=1,
        grid=(BSZ, TC_S // BS),
        in_specs=[
            pl.BlockSpec(
                (1, BS, D),
                lambda b, sb, idx: (b, (S_SC // BS) + sb, idx[b])),
            pl.BlockSpec(
                (1, BS, D),
                lambda b, sb, idx: (b, (S_SC // BS) + sb, 0)),
        ],
        out_specs=pl.BlockSpec((1, BS, D), lambda b, sb, idx: (b, sb, 0)),
    ),
    out_shape=jax.ShapeDtypeStruct((BSZ, TC_S, D), jnp.float32),
)


def kernel(inputs, adapter_outputs, adapter_outputs_copy, residual, adapter_indices):
    ao = adapter_outputs.reshape(ROWS * NA, D)
    res = residual.reshape(ROWS, D)
    idx32 = adapter_indices.astype(jnp.int32)
    idxb = jnp.broadcast_to(idx32[:, None], (BSZ, L))
    sc_out = _sc_call(ao, res, idxb)
    tc_out = _tc_call(
        idx32, adapter_outputs.reshape(BSZ, SEQ, NA * D), residual)
    return jnp.concatenate(
        [sc_out.reshape(BSZ, S_SC, D), tc_out], axis=1)


# speedup vs baseline: 3.7635x; 3.7635x over previous
import functools

import jax
import jax.numpy as jnp
from jax import lax
from jax.experimental import pallas as pl
from jax.experimental.pallas import tpu as pltpu
from jax.experimental.pallas import tpu_sc as plsc

NC, NS, L = 2, 16, 16
NW = NC * NS
BSZ, SEQ, NA, D = 2, 2048, 8, 1024
ROWS = BSZ * SEQ

S_SC = 1024
TC_S = SEQ - S_SC
SC_ROWS = BSZ * S_SC
RPW = SC_ROWS // NW
CH = 16
NBUF = 3
NCHUNK = RPW // CH
WPB = NW // BSZ
SPW = S_SC // WPB


def _body(ao, res, idxb, out, idx_v, idx_vv, *bufs_and_sems):
    gbuf = bufs_and_sems[0:NBUF]
    rbuf = bufs_and_sems[NBUF:2 * NBUF]
    gsem = bufs_and_sems[2 * NBUF:3 * NBUF]
    rsem = bufs_and_sems[3 * NBUF:4 * NBUF]
    osem = bufs_and_sems[4 * NBUF:5 * NBUF]

    c = lax.axis_index("c")
    s = lax.axis_index("s")
    wid = s * NC + c
    b = wid // WPB
    s0 = (wid % WPB) * SPW
    gbase = b * SEQ + s0
    obase = b * S_SC + s0

    pltpu.sync_copy(idxb.at[b], idx_vv)
    iv = idx_vv[...]

    for j in range(RPW // L):
        idx_v[pl.ds(j * L, L)] = (
            gbase + j * L + lax.iota(jnp.int32, L)) * NA + iv

    def issue_in(ch):
        nb = ch % NBUF
        g = pltpu.async_copy(
            ao.at[idx_v.at[pl.ds(ch * CH, CH)]], gbuf[nb], gsem[nb])
        r = pltpu.async_copy(
            res.at[pl.ds(gbase + ch * CH, CH)], rbuf[nb], rsem[nb])
        return g, r

    pend_in = {}
    pend_out = {}
    for ch in range(min(NBUF - 1, NCHUNK)):
        pend_in[ch] = issue_in(ch)
    for ch in range(NCHUNK):
        nb = ch % NBUF
        ahead = ch + NBUF - 1
        if ahead < NCHUNK:
            if ahead - NBUF in pend_out:
                pend_out.pop(ahead - NBUF).wait()
            pend_in[ahead] = issue_in(ahead)
        g, r = pend_in.pop(ch)
        g.wait()
        r.wait()

        def row_add(i, carry, _g=gbuf[nb], _r=rbuf[nb]):
            for j in range(D // L):
                sl = pl.ds(j * L, L)
                plsc.addupdate(_r.at[i, sl], _g[i, sl])
            return carry

        lax.fori_loop(0, CH, row_add, 0)
        pend_out[ch] = pltpu.async_copy(
            rbuf[nb], out.at[pl.ds(obase + ch * CH, CH)], osem[nb])
    for cp in pend_out.values():
        cp.wait()


_sc_call = functools.partial(
    pl.kernel,
    out_type=jax.ShapeDtypeStruct((SC_ROWS, D), jnp.float32),
    mesh=plsc.VectorSubcoreMesh(core_axis_name="c", subcore_axis_name="s"),
    scratch_types=(
        [pltpu.VMEM((RPW,), jnp.int32),
         pltpu.VMEM((L,), jnp.int32)]
        + [pltpu.VMEM((CH, D), jnp.float32)] * (2 * NBUF)
        + [pltpu.SemaphoreType.DMA] * (3 * NBUF)
    ),
)(_body)


BS = 256


def _tc_body(idx_ref, ao_hbm, res_hbm, o_ref, abuf, rbuf, asem, rsem):
    b = pl.program_id(0)
    sb = pl.program_id(1)
    s0 = S_SC + sb * BS
    idx = idx_ref[b]
    ca = pltpu.make_async_copy(ao_hbm.at[b, pl.ds(s0, BS), idx], abuf, asem)
    ca.start()
    cr = pltpu.make_async_copy(res_hbm.at[b, pl.ds(s0, BS)], rbuf, rsem)
    cr.start()
    ca.wait()
    cr.wait()
    o_ref[...] = (abuf[...] + rbuf[...])[None]


_tc_call = pl.pallas_call(
    _tc_body,
    grid_spec=pltpu.PrefetchScalarGridSpec(
        num_scalar_prefetch=1,
        grid=(BSZ, TC_S // BS),
        in_specs=[
            pl.BlockSpec(memory_space=pl.ANY),
            pl.BlockSpec(memory_space=pl.ANY),
        ],
        out_specs=pl.BlockSpec((1, BS, D), lambda b, sb, idx: (b, sb, 0)),
        scratch_shapes=[
            pltpu.VMEM((BS, D), jnp.float32),
            pltpu.VMEM((BS, D), jnp.float32),
            pltpu.SemaphoreType.DMA,
            pltpu.SemaphoreType.DMA,
        ],
    ),
    out_shape=jax.ShapeDtypeStruct((BSZ, TC_S, D), jnp.float32),
)


def kernel(inputs, adapter_outputs, adapter_outputs_copy, residual, adapter_indices):
    ao = adapter_outputs.reshape(ROWS * NA, D)
    res = residual.reshape(ROWS, D)
    idx32 = adapter_indices.astype(jnp.int32)
    idxb = jnp.broadcast_to(idx32[:, None], (BSZ, L))
    sc_part = _sc_call(ao, res, idxb).reshape(BSZ, S_SC, D)
    tc_part = _tc_call(idx32, adapter_outputs, residual)
    return jnp.concatenate([sc_part, tc_part], axis=1)


# speedup vs baseline: 4.2204x; 1.1214x over previous
import functools

import jax
import jax.numpy as jnp
from jax import lax
from jax.experimental import pallas as pl
from jax.experimental.pallas import tpu as pltpu
from jax.experimental.pallas import tpu_sc as plsc

NC, NS, L = 2, 16, 16
NW = NC * NS
BSZ, SEQ, NA, D = 2, 2048, 8, 1024
ROWS = BSZ * SEQ

S_SC = 1024
TC_S = SEQ - S_SC
SC_ROWS = BSZ * S_SC
RPW = SC_ROWS // NW
CH = 16
NBUF = 3
NCHUNK = RPW // CH
WPB = NW // BSZ
SPW = S_SC // WPB


def _body(ao, res, idxb, out, idx_v, idx_vv, *bufs_and_sems):
    gbuf = bufs_and_sems[0:NBUF]
    rbuf = bufs_and_sems[NBUF:2 * NBUF]
    gsem = bufs_and_sems[2 * NBUF:3 * NBUF]
    rsem = bufs_and_sems[3 * NBUF:4 * NBUF]
    osem = bufs_and_sems[4 * NBUF:5 * NBUF]

    c = lax.axis_index("c")
    s = lax.axis_index("s")
    wid = s * NC + c
    b = wid // WPB
    s0 = (wid % WPB) * SPW
    gbase = b * SEQ + s0
    obase = gbase

    pltpu.sync_copy(idxb.at[b], idx_vv)
    iv = idx_vv[...]

    for j in range(RPW // L):
        idx_v[pl.ds(j * L, L)] = (
            gbase + j * L + lax.iota(jnp.int32, L)) * NA + iv

    def issue_in(ch):
        nb = ch % NBUF
        g = pltpu.async_copy(
            ao.at[idx_v.at[pl.ds(ch * CH, CH)]], gbuf[nb], gsem[nb])
        r = pltpu.async_copy(
            res.at[pl.ds(gbase + ch * CH, CH)], rbuf[nb], rsem[nb])
        return g, r

    pend_in = {}
    pend_out = {}
    for ch in range(min(NBUF - 1, NCHUNK)):
        pend_in[ch] = issue_in(ch)
    for ch in range(NCHUNK):
        nb = ch % NBUF
        ahead = ch + NBUF - 1
        if ahead < NCHUNK:
            if ahead - NBUF in pend_out:
                pend_out.pop(ahead - NBUF).wait()
            pend_in[ahead] = issue_in(ahead)
        g, r = pend_in.pop(ch)
        g.wait()
        r.wait()

        def row_add(i, carry, _g=gbuf[nb], _r=rbuf[nb]):
            for j in range(D // L):
                sl = pl.ds(j * L, L)
                plsc.addupdate(_r.at[i, sl], _g[i, sl])
            return carry

        lax.fori_loop(0, CH, row_add, 0)
        pend_out[ch] = pltpu.async_copy(
            rbuf[nb], out.at[pl.ds(obase + ch * CH, CH)], osem[nb])
    for cp in pend_out.values():
        cp.wait()


_sc_call = functools.partial(
    pl.kernel,
    out_type=jax.ShapeDtypeStruct((ROWS, D), jnp.float32),
    mesh=plsc.VectorSubcoreMesh(core_axis_name="c", subcore_axis_name="s"),
    scratch_types=(
        [pltpu.VMEM((RPW,), jnp.int32),
         pltpu.VMEM((L,), jnp.int32)]
        + [pltpu.VMEM((CH, D), jnp.float32)] * (2 * NBUF)
        + [pltpu.SemaphoreType.DMA] * (3 * NBUF)
    ),
)(_body)


BS = 256


NSB = TC_S // BS


def _tc_body(idx_ref, ao_hbm, res_hbm, o_ref, abuf, rbuf, asem, rsem):
    b = pl.program_id(0)
    sb = pl.program_id(1)
    step = b * NSB + sb

    def copies(stp, slot):
        bb = stp // NSB
        s0 = S_SC + (stp % NSB) * BS
        ca = pltpu.make_async_copy(
            ao_hbm.at[bb, pl.ds(s0, BS), idx_ref[bb]], abuf.at[slot],
            asem.at[slot])
        cr = pltpu.make_async_copy(
            res_hbm.at[bb, pl.ds(s0, BS)], rbuf.at[slot], rsem.at[slot])
        return ca, cr

    @pl.when(step == 0)
    def _():
        ca, cr = copies(step, 0)
        ca.start()
        cr.start()

    @pl.when(step + 1 < BSZ * NSB)
    def _():
        ca, cr = copies(step + 1, (step + 1) % 2)
        ca.start()
        cr.start()

    slot = step % 2
    ca, cr = copies(step, slot)
    ca.wait()
    cr.wait()
    o_ref[...] = (abuf[slot] + rbuf[slot])[None]


_tc_call = pl.pallas_call(
    _tc_body,
    grid_spec=pltpu.PrefetchScalarGridSpec(
        num_scalar_prefetch=1,
        grid=(BSZ, NSB),
        in_specs=[
            pl.BlockSpec(memory_space=pl.ANY),
            pl.BlockSpec(memory_space=pl.ANY),
        ],
        out_specs=pl.BlockSpec((1, BS, D), lambda b, sb, idx: (b, sb, 0)),
        scratch_shapes=[
            pltpu.VMEM((2, BS, D), jnp.float32),
            pltpu.VMEM((2, BS, D), jnp.float32),
            pltpu.SemaphoreType.DMA((2,)),
            pltpu.SemaphoreType.DMA((2,)),
        ],
    ),
    out_shape=jax.ShapeDtypeStruct((BSZ, TC_S, D), jnp.float32),
)


def kernel(inputs, adapter_outputs, adapter_outputs_copy, residual, adapter_indices):
    ao = adapter_outputs.reshape(ROWS * NA, D)
    res = residual.reshape(ROWS, D)
    idx32 = adapter_indices.astype(jnp.int32)
    idxb = jnp.broadcast_to(idx32[:, None], (BSZ, L))
    sc_full = _sc_call(ao, res, idxb).reshape(BSZ, SEQ, D)
    tc_part = _tc_call(idx32, adapter_outputs, residual)
    return lax.dynamic_update_slice(sc_full, tc_part, (0, S_SC, 0))


# speedup vs baseline: 4.3378x; 1.0278x over previous
import functools

import jax
import jax.numpy as jnp
from jax import lax
from jax.experimental import pallas as pl
from jax.experimental.pallas import tpu as pltpu
from jax.experimental.pallas import tpu_sc as plsc

NC, NS, L = 2, 16, 16
NW = NC * NS
BSZ, SEQ, NA, D = 2, 2048, 8, 1024
ROWS = BSZ * SEQ

S_SC = 1536
TC_S = SEQ - S_SC
SC_ROWS = BSZ * S_SC
RPW = SC_ROWS // NW
CH = 16
NBUF = 3
NCHUNK = RPW // CH
WPB = NW // BSZ
SPW = S_SC // WPB


def _body(ao, res, idxb, out, idx_v, idx_vv, *bufs_and_sems):
    gbuf = bufs_and_sems[0:NBUF]
    rbuf = bufs_and_sems[NBUF:2 * NBUF]
    gsem = bufs_and_sems[2 * NBUF:3 * NBUF]
    rsem = bufs_and_sems[3 * NBUF:4 * NBUF]
    osem = bufs_and_sems[4 * NBUF:5 * NBUF]

    c = lax.axis_index("c")
    s = lax.axis_index("s")
    wid = s * NC + c
    b = wid // WPB
    s0 = (wid % WPB) * SPW
    gbase = b * SEQ + s0
    obase = gbase

    pltpu.sync_copy(idxb.at[b], idx_vv)
    iv = idx_vv[...]

    for j in range(RPW // L):
        idx_v[pl.ds(j * L, L)] = (
            gbase + j * L + lax.iota(jnp.int32, L)) * NA + iv

    def issue_in(ch):
        nb = ch % NBUF
        g = pltpu.async_copy(
            ao.at[idx_v.at[pl.ds(ch * CH, CH)]], gbuf[nb], gsem[nb])
        r = pltpu.async_copy(
            res.at[pl.ds(gbase + ch * CH, CH)], rbuf[nb], rsem[nb])
        return g, r

    pend_in = {}
    pend_out = {}
    for ch in range(min(NBUF - 1, NCHUNK)):
        pend_in[ch] = issue_in(ch)
    for ch in range(NCHUNK):
        nb = ch % NBUF
        ahead = ch + NBUF - 1
        if ahead < NCHUNK:
            if ahead - NBUF in pend_out:
                pend_out.pop(ahead - NBUF).wait()
            pend_in[ahead] = issue_in(ahead)
        g, r = pend_in.pop(ch)
        g.wait()
        r.wait()

        def row_add(i, carry, _g=gbuf[nb], _r=rbuf[nb]):
            for j in range(D // L):
                sl = pl.ds(j * L, L)
                plsc.addupdate(_r.at[i, sl], _g[i, sl])
            return carry

        lax.fori_loop(0, CH, row_add, 0)
        pend_out[ch] = pltpu.async_copy(
            rbuf[nb], out.at[pl.ds(obase + ch * CH, CH)], osem[nb])
    for cp in pend_out.values():
        cp.wait()


_sc_call = functools.partial(
    pl.kernel,
    out_type=jax.ShapeDtypeStruct((ROWS, D), jnp.float32),
    mesh=plsc.VectorSubcoreMesh(core_axis_name="c", subcore_axis_name="s"),
    scratch_types=(
        [pltpu.VMEM((RPW,), jnp.int32),
         pltpu.VMEM((L,), jnp.int32)]
        + [pltpu.VMEM((CH, D), jnp.float32)] * (2 * NBUF)
        + [pltpu.SemaphoreType.DMA] * (3 * NBUF)
    ),
)(_body)


BS = 256


NSB = TC_S // BS


def _tc_body(idx_ref, ao_hbm, res_hbm, o_ref, abuf, rbuf, asem, rsem):
    b = pl.program_id(0)
    sb = pl.program_id(1)
    step = b * NSB + sb

    def copies(stp, slot):
        bb = stp // NSB
        s0 = S_SC + (stp % NSB) * BS
        ca = pltpu.make_async_copy(
            ao_hbm.at[bb, pl.ds(s0, BS), idx_ref[bb]], abuf.at[slot],
            asem.at[slot])
        cr = pltpu.make_async_copy(
            res_hbm.at[bb, pl.ds(s0, BS)], rbuf.at[slot], rsem.at[slot])
        return ca, cr

    @pl.when(step == 0)
    def _():
        ca, cr = copies(step, 0)
        ca.start()
        cr.start()

    @pl.when(step + 1 < BSZ * NSB)
    def _():
        ca, cr = copies(step + 1, (step + 1) % 2)
        ca.start()
        cr.start()

    slot = step % 2
    ca, cr = copies(step, slot)
    ca.wait()
    cr.wait()
    o_ref[...] = (abuf[slot] + rbuf[slot])[None]


_tc_call = pl.pallas_call(
    _tc_body,
    grid_spec=pltpu.PrefetchScalarGridSpec(
        num_scalar_prefetch=1,
        grid=(BSZ, NSB),
        in_specs=[
            pl.BlockSpec(memory_space=pl.ANY),
            pl.BlockSpec(memory_space=pl.ANY),
        ],
        out_specs=pl.BlockSpec((1, BS, D), lambda b, sb, idx: (b, sb, 0)),
        scratch_shapes=[
            pltpu.VMEM((2, BS, D), jnp.float32),
            pltpu.VMEM((2, BS, D), jnp.float32),
            pltpu.SemaphoreType.DMA((2,)),
            pltpu.SemaphoreType.DMA((2,)),
        ],
    ),
    out_shape=jax.ShapeDtypeStruct((BSZ, TC_S, D), jnp.float32),
)


def kernel(inputs, adapter_outputs, adapter_outputs_copy, residual, adapter_indices):
    ao = adapter_outputs.reshape(ROWS * NA, D)
    res = residual.reshape(ROWS, D)
    idx32 = adapter_indices.astype(jnp.int32)
    idxb = jnp.broadcast_to(idx32[:, None], (BSZ, L))
    sc_full = _sc_call(ao, res, idxb).reshape(BSZ, SEQ, D)
    tc_part = _tc_call(idx32, adapter_outputs, residual)
    return lax.dynamic_update_slice(sc_full, tc_part, (0, S_SC, 0))


# speedup vs baseline: 4.5997x; 1.0604x over previous
import functools

import jax
import jax.numpy as jnp
from jax import lax
from jax.experimental import pallas as pl
from jax.experimental.pallas import tpu as pltpu
from jax.experimental.pallas import tpu_sc as plsc

NC, NS, L = 2, 16, 16
NW = NC * NS
BSZ, SEQ, NA, D = 2, 2048, 8, 1024
ROWS = BSZ * SEQ
RPW = ROWS // NW
CH = 16
NBUF = 3
NCHUNK = RPW // CH
WPB = NW // BSZ


def _body(ao, res, idxb, out, idx_v, idx_vv, *bufs_and_sems):
    gbuf = bufs_and_sems[0:NBUF]
    rbuf = bufs_and_sems[NBUF:2 * NBUF]
    gsem = bufs_and_sems[2 * NBUF:3 * NBUF]
    rsem = bufs_and_sems[3 * NBUF:4 * NBUF]
    osem = bufs_and_sems[4 * NBUF:5 * NBUF]

    c = lax.axis_index("c")
    s = lax.axis_index("s")
    wid = s * NC + c
    b = wid // WPB
    base = wid * RPW

    pltpu.sync_copy(idxb.at[b], idx_vv)
    iv = idx_vv[...]

    def build_idx(ch):
        for j in range(CH // L):
            o = ch * CH + j * L
            idx_v[pl.ds(o, L)] = (
                base + o + lax.iota(jnp.int32, L)) * NA + iv

    def issue_in(ch):
        nb = ch % NBUF
        g = pltpu.async_copy(
            ao.at[idx_v.at[pl.ds(ch * CH, CH)]], gbuf[nb], gsem[nb])
        r = pltpu.async_copy(
            res.at[pl.ds(base + ch * CH, CH)], rbuf[nb], rsem[nb])
        return g, r

    pend_in = {}
    pend_out = {}
    for ch in range(NBUF - 1):
        build_idx(ch)
        pend_in[ch] = issue_in(ch)
    for ch in range(NBUF - 1, NCHUNK):
        build_idx(ch)

    for ch in range(NCHUNK):
        nb = ch % NBUF
        ahead = ch + NBUF - 1
        if ahead < NCHUNK:
            if ahead - NBUF in pend_out:
                pend_out.pop(ahead - NBUF).wait()
            pend_in[ahead] = issue_in(ahead)
        g, r = pend_in.pop(ch)
        g.wait()
        r.wait()

        def row_add(i, carry, _g=gbuf[nb], _r=rbuf[nb]):
            for j in range(D // L):
                sl = pl.ds(j * L, L)
                plsc.addupdate(_r.at[i, sl], _g[i, sl])
            return carry

        lax.fori_loop(0, CH, row_add, 0)
        pend_out[ch] = pltpu.async_copy(
            rbuf[nb], out.at[pl.ds(base + ch * CH, CH)], osem[nb])
    for cp in pend_out.values():
        cp.wait()


_sc_call = functools.partial(
    pl.kernel,
    out_type=jax.ShapeDtypeStruct((ROWS, D), jnp.float32),
    mesh=plsc.VectorSubcoreMesh(core_axis_name="c", subcore_axis_name="s"),
    scratch_types=(
        [pltpu.VMEM((RPW,), jnp.int32),
         pltpu.VMEM((L,), jnp.int32)]
        + [pltpu.VMEM((CH, D), jnp.float32)] * (2 * NBUF)
        + [pltpu.SemaphoreType.DMA] * (3 * NBUF)
    ),
)(_body)


def kernel(inputs, adapter_outputs, adapter_outputs_copy, residual, adapter_indices):
    ao = adapter_outputs.reshape(ROWS * NA, D)
    res = residual.reshape(ROWS, D)
    idxb = jnp.broadcast_to(
        adapter_indices.astype(jnp.int32)[:, None], (BSZ, L))
    out = _sc_call(ao, res, idxb)
    return out.reshape(BSZ, SEQ, D)
